# baseline (speedup 1.0000x reference)
import functools

import jax
import jax.numpy as jnp
from jax import lax
from jax.experimental import pallas as pl
from jax.experimental.pallas import tpu as pltpu
from jax.experimental.pallas import tpu_sc as plsc


def _make_deg_kernel(n, e, nc, ns):
    nw = nc * ns
    per_w = e // nw
    n_vecs = per_w // 16
    n_zero = n // 16

    mesh = plsc.VectorSubcoreMesh(core_axis_name="c", subcore_axis_name="s")

    @functools.partial(
        pl.kernel,
        mesh=mesh,
        out_type=jax.ShapeDtypeStruct((nw, n), jnp.float32),
        scratch_types=[
            pltpu.VMEM((per_w,), jnp.int32),
            pltpu.VMEM((n,), jnp.float32),
        ],
        compiler_params=pltpu.CompilerParams(needs_layout_passes=False),
    )
    def deg_kernel(dst_hbm, out_hbm, dst_v, hist_v):
        c = lax.axis_index("c")
        s = lax.axis_index("s")
        wid = s * nc + c
        pltpu.sync_copy(dst_hbm.at[wid], dst_v)

        def zero_body(i, carry):
            hist_v[pl.ds(i * 16, 16)] = jnp.zeros((16,), jnp.float32)
            return carry

        lax.fori_loop(0, n_zero, zero_body, 0)

        ones = jnp.ones((16,), jnp.float32)

        def acc_body(i, carry):
            idx = dst_v[pl.ds(i * 16, 16)]
            plsc.addupdate_scatter(hist_v, [idx], ones)
            return carry

        lax.fori_loop(0, n_vecs, acc_body, 0)

        pltpu.sync_copy(hist_v, out_hbm.at[wid])

    return deg_kernel


def _make_agg_kernel(n, n_pad, e_pad, d, nc, ns, chunk):
    nw = nc * ns
    per_w = e_pad // nw
    k = per_w // chunk
    rows_per_tile = n // ns

    mesh = plsc.VectorSubcoreMesh(core_axis_name="c", subcore_axis_name="s")

    @functools.partial(
        pl.kernel,
        mesh=mesh,
        out_type=jax.ShapeDtypeStruct((nc, n, d), jnp.float32),
        scratch_types=[
            pltpu.VMEM((k, chunk), jnp.int32),
            pltpu.VMEM((k, chunk), jnp.int32),
            pltpu.VMEM((chunk, d), jnp.float32),
            pltpu.SemaphoreType.DMA,
            pltpu.SemaphoreType.DMA,
            pltpu.VMEM_SHARED((n_pad, d), jnp.float32),
        ],
        compiler_params=pltpu.CompilerParams(
            needs_layout_passes=False, use_tc_tiling_on_sc=False),
    )
    def agg_kernel(g_hbm, src_hbm, dst_hbm, zeros_hbm, out_hbm,
                   src_v, dst_v, buf, gsem, ssem, acc_sh):
        c = lax.axis_index("c")
        s = lax.axis_index("s")
        wid = s * nc + c
        r0 = s * rows_per_tile

        pltpu.sync_copy(src_hbm.at[wid], src_v)
        pltpu.sync_copy(dst_hbm.at[wid], dst_v)

        @pl.when(c == 0)
        def _():
            pltpu.sync_copy(g_hbm.at[pl.ds(r0, rows_per_tile)],
                            acc_sh.at[pl.ds(r0, rows_per_tile)])

        @pl.when(c != 0)
        def _():
            pltpu.sync_copy(zeros_hbm, acc_sh.at[pl.ds(r0, rows_per_tile)])

        plsc.subcore_barrier()

        def body(c, carry):
            pltpu.async_copy(g_hbm.at[src_v.at[c]], buf, gsem).wait()
            pltpu.async_copy(
                buf, acc_sh.at[dst_v.at[c]], ssem, add=True).wait()
            return carry

        lax.fori_loop(0, k, body, 0)

        plsc.subcore_barrier()

        pltpu.sync_copy(acc_sh.at[pl.ds(r0, rows_per_tile)],
                        out_hbm.at[c, pl.ds(r0, rows_per_tile)])

    return agg_kernel


def _tc_pre(deg_parts_t, x, w1, n_pad):
    n, d_in = x.shape
    d_hid = w1.shape[1]

    def body(deg_ref, x_ref, w_ref, g_ref, dinv_ref):
        deg = 1.0 + jnp.sum(deg_ref[...], axis=1, keepdims=True)
        dinv = lax.rsqrt(deg)
        h = jnp.dot(x_ref[...], w_ref[...], preferred_element_type=jnp.float32)
        g_ref[0:n, :] = h * dinv
        g_ref[n:n_pad, :] = jnp.zeros((n_pad - n, d_hid), jnp.float32)
        dinv_ref[...] = dinv

    return pl.pallas_call(
        body,
        out_shape=(
            jax.ShapeDtypeStruct((n_pad, d_hid), jnp.float32),
            jax.ShapeDtypeStruct((n, 1), jnp.float32),
        ),
    )(deg_parts_t, x, w1)


def _tc_mid(p, dinv, b1, w2, n_pad):
    _, n, d_hid = p.shape
    d_in = w2.shape[1]

    def body(p_ref, dinv_ref, b_ref, w_ref, g2_ref):
        dinv = dinv_ref[...]
        z = jnp.maximum((p_ref[0] + p_ref[1]) * dinv + b_ref[...], 0.0)
        h = jnp.dot(z, w_ref[...], preferred_element_type=jnp.float32)
        g2_ref[0:n, :] = h * dinv
        g2_ref[n:n_pad, :] = jnp.zeros((n_pad - n, d_in), jnp.float32)

    return pl.pallas_call(
        body,
        out_shape=jax.ShapeDtypeStruct((n_pad, d_in), jnp.float32),
    )(p, dinv, b1, w2)


def _tc_post(p, dinv, b2):
    _, n, d = p.shape

    def body(p_ref, dinv_ref, b_ref, out_ref):
        out_ref[...] = jnp.maximum(
            (p_ref[0] + p_ref[1]) * dinv_ref[...] + b_ref[...], 0.0)

    return pl.pallas_call(
        body,
        out_shape=jax.ShapeDtypeStruct((n, d), jnp.float32),
    )(p, dinv, b2)


def kernel(x, edge_index, W1, b1, W2, b2):
    n, d_in = x.shape
    d_hid = W1.shape[1]
    e = edge_index.shape[1]

    info = plsc.get_sparse_core_info()
    nc, ns = info.num_cores, info.num_subcores
    nw = nc * ns
    per_w = e // nw
    assert per_w * nw == e

    chunk = 128
    per_w_pad = -(-per_w // chunk) * chunk
    e_pad = per_w_pad * nw
    n_pad = n + 16

    ei = edge_index.astype(jnp.int32)
    pad = jnp.full((e_pad - e,), n, jnp.int32)
    src_r = jnp.concatenate([ei[0], pad]).reshape(nw, per_w_pad // chunk,
                                                  chunk)
    dst_r = jnp.concatenate([ei[1], pad]).reshape(nw, per_w_pad // chunk,
                                                  chunk)
    dst_flat = ei[1].reshape(nw, per_w)

    deg_parts = _make_deg_kernel(n, e, nc, ns)(dst_flat)
    g1, dinv = _tc_pre(deg_parts.T, x, W1, n_pad)

    zeros1 = jnp.zeros((n // ns, d_hid), jnp.float32)
    p1 = _make_agg_kernel(n, n_pad, e_pad, d_hid, nc, ns, chunk)(
        g1, src_r, dst_r, zeros1)

    g2 = _tc_mid(p1, dinv, b1.reshape(1, d_hid), W2, n_pad)

    zeros2 = jnp.zeros((n // ns, d_in), jnp.float32)
    p2 = _make_agg_kernel(n, n_pad, e_pad, d_in, nc, ns, chunk)(
        g2, src_r, dst_r, zeros2)

    return _tc_post(p2, dinv, b2.reshape(1, d_in))


# speedup vs baseline: 1.7090x; 1.7090x over previous
import functools

import jax
import jax.numpy as jnp
from jax import lax
from jax.experimental import pallas as pl
from jax.experimental.pallas import tpu as pltpu
from jax.experimental.pallas import tpu_sc as plsc


def _make_deg_kernel(n, e, nc, ns):
    nw = nc * ns
    per_w = e // nw
    n_vecs = per_w // 16
    n_zero = n // 16

    mesh = plsc.VectorSubcoreMesh(core_axis_name="c", subcore_axis_name="s")

    @functools.partial(
        pl.kernel,
        mesh=mesh,
        out_type=jax.ShapeDtypeStruct((nw, n), jnp.float32),
        scratch_types=[
            pltpu.VMEM((per_w,), jnp.int32),
            pltpu.VMEM((n,), jnp.float32),
        ],
        compiler_params=pltpu.CompilerParams(needs_layout_passes=False),
    )
    def deg_kernel(dst_hbm, out_hbm, dst_v, hist_v):
        c = lax.axis_index("c")
        s = lax.axis_index("s")
        wid = s * nc + c
        pltpu.sync_copy(dst_hbm.at[wid], dst_v)

        def zero_body(i, carry):
            hist_v[pl.ds(i * 16, 16)] = jnp.zeros((16,), jnp.float32)
            return carry

        lax.fori_loop(0, n_zero, zero_body, 0)

        ones = jnp.ones((16,), jnp.float32)

        def acc_body(i, carry):
            idx = dst_v[pl.ds(i * 16, 16)]
            plsc.addupdate_scatter(hist_v, [idx], ones)
            return carry

        lax.fori_loop(0, n_vecs, acc_body, 0)

        pltpu.sync_copy(hist_v, out_hbm.at[wid])

    return deg_kernel


def _make_agg_kernel(n, n_pad, e_pad, d, nc, ns, chunk):
    nw = nc * ns
    per_w = e_pad // nw
    k = per_w // chunk
    rows_per_tile = n // ns

    mesh = plsc.VectorSubcoreMesh(core_axis_name="c", subcore_axis_name="s")

    @functools.partial(
        pl.kernel,
        mesh=mesh,
        out_type=jax.ShapeDtypeStruct((nc, n, d), jnp.float32),
        scratch_types=[
            pltpu.VMEM((k, chunk), jnp.int32),
            pltpu.VMEM((k, chunk), jnp.int32),
            pltpu.VMEM((chunk, d), jnp.float32),
            pltpu.SemaphoreType.DMA,
            pltpu.SemaphoreType.DMA,
            pltpu.VMEM_SHARED((n_pad, d), jnp.float32),
        ],
        compiler_params=pltpu.CompilerParams(
            needs_layout_passes=False, use_tc_tiling_on_sc=False),
    )
    def agg_kernel(g_hbm, src_hbm, dst_hbm, zeros_hbm, out_hbm,
                   src_v, dst_v, buf, gsem, ssem, acc_sh):
        c = lax.axis_index("c")
        s = lax.axis_index("s")
        wid = s * nc + c
        r0 = s * rows_per_tile

        pltpu.sync_copy(src_hbm.at[wid], src_v)
        pltpu.sync_copy(dst_hbm.at[wid], dst_v)

        @pl.when(c == 0)
        def _():
            pltpu.sync_copy(g_hbm.at[pl.ds(r0, rows_per_tile)],
                            acc_sh.at[pl.ds(r0, rows_per_tile)])

        @pl.when(c != 0)
        def _():
            pltpu.sync_copy(zeros_hbm, acc_sh.at[pl.ds(r0, rows_per_tile)])

        plsc.subcore_barrier()

        def body(c, carry):
            pltpu.async_copy(g_hbm.at[src_v.at[c]], buf, gsem).wait()
            pltpu.async_copy(
                buf, acc_sh.at[dst_v.at[c]], ssem, add=True).wait()
            return carry

        lax.fori_loop(0, k, body, 0)

        plsc.subcore_barrier()

        pltpu.sync_copy(acc_sh.at[pl.ds(r0, rows_per_tile)],
                        out_hbm.at[c, pl.ds(r0, rows_per_tile)])

    return agg_kernel


def _tc_pre(deg_parts_t, x, w1, n_pad):
    n, d_in = x.shape
    d_hid = w1.shape[1]

    def body(deg_ref, x_ref, w_ref, g_ref, dinv_ref):
        deg = 1.0 + jnp.sum(deg_ref[...], axis=1, keepdims=True)
        dinv = lax.rsqrt(deg)
        h = jnp.dot(x_ref[...], w_ref[...], preferred_element_type=jnp.float32)
        g_ref[0:n, :] = h * dinv
        g_ref[n:n_pad, :] = jnp.zeros((n_pad - n, d_hid), jnp.float32)
        dinv_ref[...] = dinv

    return pl.pallas_call(
        body,
        out_shape=(
            jax.ShapeDtypeStruct((n_pad, d_hid), jnp.float32),
            jax.ShapeDtypeStruct((n, 1), jnp.float32),
        ),
    )(deg_parts_t, x, w1)


def _tc_mid(p, dinv, b1, w2, n_pad):
    _, n, d_hid = p.shape
    d_in = w2.shape[1]

    def body(p_ref, dinv_ref, b_ref, w_ref, g2_ref):
        dinv = dinv_ref[...]
        z = jnp.maximum((p_ref[0] + p_ref[1]) * dinv + b_ref[...], 0.0)
        h = jnp.dot(z, w_ref[...], preferred_element_type=jnp.float32)
        g2_ref[0:n, :] = h * dinv
        g2_ref[n:n_pad, :] = jnp.zeros((n_pad - n, d_in), jnp.float32)

    return pl.pallas_call(
        body,
        out_shape=jax.ShapeDtypeStruct((n_pad, d_in), jnp.float32),
    )(p, dinv, b1, w2)


def _tc_post(p, dinv, b2):
    _, n, d = p.shape

    def body(p_ref, dinv_ref, b_ref, out_ref):
        out_ref[...] = jnp.maximum(
            (p_ref[0] + p_ref[1]) * dinv_ref[...] + b_ref[...], 0.0)

    return pl.pallas_call(
        body,
        out_shape=jax.ShapeDtypeStruct((n, d), jnp.float32),
    )(p, dinv, b2)


def kernel(x, edge_index, W1, b1, W2, b2):
    n, d_in = x.shape
    d_hid = W1.shape[1]
    e = edge_index.shape[1]

    info = plsc.get_sparse_core_info()
    nc, ns = info.num_cores, info.num_subcores
    nw = nc * ns
    per_w = e // nw
    assert per_w * nw == e

    chunk = 128
    per_w_pad = -(-per_w // chunk) * chunk
    e_pad = per_w_pad * nw
    n_pad = n + 128

    ei = edge_index.astype(jnp.int32)
    pad = n + (jnp.arange(e_pad - e, dtype=jnp.int32) % 128)
    src_r = jnp.concatenate([ei[0], pad]).reshape(nw, per_w_pad // chunk,
                                                  chunk)
    dst_r = jnp.concatenate([ei[1], pad]).reshape(nw, per_w_pad // chunk,
                                                  chunk)
    dst_flat = ei[1].reshape(nw, per_w)

    deg_parts = _make_deg_kernel(n, e, nc, ns)(dst_flat)
    g1, dinv = _tc_pre(deg_parts.T, x, W1, n_pad)

    zeros1 = jnp.zeros((n // ns, d_hid), jnp.float32)
    p1 = _make_agg_kernel(n, n_pad, e_pad, d_hid, nc, ns, chunk)(
        g1, src_r, dst_r, zeros1)

    g2 = _tc_mid(p1, dinv, b1.reshape(1, d_hid), W2, n_pad)

    zeros2 = jnp.zeros((n // ns, d_in), jnp.float32)
    p2 = _make_agg_kernel(n, n_pad, e_pad, d_in, nc, ns, chunk)(
        g2, src_r, dst_r, zeros2)

    return _tc_post(p2, dinv, b2.reshape(1, d_in))


# speedup vs baseline: 2.0656x; 1.2086x over previous
import functools

import jax
import jax.numpy as jnp
from jax import lax
from jax.experimental import pallas as pl
from jax.experimental.pallas import tpu as pltpu
from jax.experimental.pallas import tpu_sc as plsc


def _make_deg_kernel(n, e, nc, ns):
    nw = nc * ns
    per_w = e // nw
    n_vecs = per_w // 16
    n_zero = n // 16

    mesh = plsc.VectorSubcoreMesh(core_axis_name="c", subcore_axis_name="s")

    @functools.partial(
        pl.kernel,
        mesh=mesh,
        out_type=jax.ShapeDtypeStruct((nw, n), jnp.float32),
        scratch_types=[
            pltpu.VMEM((per_w,), jnp.int32),
            pltpu.VMEM((n,), jnp.float32),
        ],
        compiler_params=pltpu.CompilerParams(needs_layout_passes=False),
    )
    def deg_kernel(dst_hbm, out_hbm, dst_v, hist_v):
        c = lax.axis_index("c")
        s = lax.axis_index("s")
        wid = s * nc + c
        pltpu.sync_copy(dst_hbm.at[wid], dst_v)

        def zero_body(i, carry):
            hist_v[pl.ds(i * 16, 16)] = jnp.zeros((16,), jnp.float32)
            return carry

        lax.fori_loop(0, n_zero, zero_body, 0)

        ones = jnp.ones((16,), jnp.float32)

        def acc_body(i, carry):
            idx = dst_v[pl.ds(i * 16, 16)]
            plsc.addupdate_scatter(hist_v, [idx], ones)
            return carry

        lax.fori_loop(0, n_vecs, acc_body, 0)

        pltpu.sync_copy(hist_v, out_hbm.at[wid])

    return deg_kernel


def _make_agg_kernel(n, n_pad, e_pad, d, nc, ns, chunk, overlap):
    nw = nc * ns
    per_w = e_pad // nw
    k = per_w // chunk
    rows_per_tile = n // ns

    mesh = plsc.VectorSubcoreMesh(core_axis_name="c", subcore_axis_name="s")

    @functools.partial(
        pl.kernel,
        mesh=mesh,
        out_type=jax.ShapeDtypeStruct((nc, n, d), jnp.float32),
        scratch_types=[
            pltpu.VMEM((k // 2, chunk), jnp.int32),
            pltpu.VMEM((k // 2, chunk), jnp.int32),
            [pltpu.VMEM((chunk, d), jnp.float32)
             for _ in range(2 if overlap else 1)],
            pltpu.SemaphoreType.DMA,
            pltpu.SemaphoreType.DMA,
            pltpu.VMEM_SHARED((n_pad, d), jnp.float32),
        ],
        compiler_params=pltpu.CompilerParams(
            needs_layout_passes=False, use_tc_tiling_on_sc=False),
    )
    def agg_kernel(g_hbm, src_hbm, dst_hbm, zeros_hbm, out_hbm,
                   src_v, dst_v, bufs, gsem, ssem, acc_sh):
        c = lax.axis_index("c")
        s = lax.axis_index("s")
        wid = s * nc + c
        r0 = s * rows_per_tile

        @pl.when(c == 0)
        def _():
            pltpu.sync_copy(g_hbm.at[pl.ds(r0, rows_per_tile)],
                            acc_sh.at[pl.ds(r0, rows_per_tile)])

        @pl.when(c != 0)
        def _():
            pltpu.sync_copy(zeros_hbm, acc_sh.at[pl.ds(r0, rows_per_tile)])

        plsc.subcore_barrier()

        def gather(c, buf):
            pltpu.async_copy(g_hbm.at[src_v.at[c]], buf, gsem).wait()

        def scatter_start(c, buf):
            return pltpu.async_copy(
                buf, acc_sh.at[dst_v.at[c]], ssem, add=True)

        assert k % 4 == 0
        k2 = k // 2
        for h in range(2):
            pltpu.sync_copy(src_hbm.at[wid, pl.ds(h * k2, k2)], src_v)
            pltpu.sync_copy(dst_hbm.at[wid, pl.ds(h * k2, k2)], dst_v)

            if overlap:
                blk = next(bb for bb in (20, 16, 8, 4, 2)
                           if k2 % bb == 0)
                assert k2 % blk == 0

                def body(p, carry):
                    c0 = p * blk
                    hnd = None
                    for j in range(blk):
                        buf = bufs[j % 2]
                        gather(c0 + j, buf)
                        if hnd is not None:
                            hnd.wait()
                        hnd = scatter_start(c0 + j, buf)
                    hnd.wait()
                    return carry

                lax.fori_loop(0, k2 // blk, body, 0)
            else:

                def body(c, carry):
                    gather(c, bufs[0])
                    scatter_start(c, bufs[0]).wait()
                    return carry

                lax.fori_loop(0, k2, body, 0)

        plsc.subcore_barrier()

        pltpu.sync_copy(acc_sh.at[pl.ds(r0, rows_per_tile)],
                        out_hbm.at[c, pl.ds(r0, rows_per_tile)])

    return agg_kernel


def _tc_pre(deg_parts_t, x, w1, n_pad):
    n, d_in = x.shape
    d_hid = w1.shape[1]

    def body(deg_ref, x_ref, w_ref, g_ref, dinv_ref):
        deg = 1.0 + jnp.sum(deg_ref[...], axis=1, keepdims=True)
        dinv = lax.rsqrt(deg)
        h = jnp.dot(x_ref[...], w_ref[...], preferred_element_type=jnp.float32)
        g_ref[0:n, :] = h * dinv
        g_ref[n:n_pad, :] = jnp.zeros((n_pad - n, d_hid), jnp.float32)
        dinv_ref[...] = dinv

    return pl.pallas_call(
        body,
        out_shape=(
            jax.ShapeDtypeStruct((n_pad, d_hid), jnp.float32),
            jax.ShapeDtypeStruct((n, 1), jnp.float32),
        ),
    )(deg_parts_t, x, w1)


def _tc_mid(p, dinv, b1, w2, n_pad):
    _, n, d_hid = p.shape
    d_in = w2.shape[1]

    def body(p_ref, dinv_ref, b_ref, w_ref, g2_ref):
        dinv = dinv_ref[...]
        z = jnp.maximum((p_ref[0] + p_ref[1]) * dinv + b_ref[...], 0.0)
        h = jnp.dot(z, w_ref[...], preferred_element_type=jnp.float32)
        g2_ref[0:n, :] = h * dinv
        g2_ref[n:n_pad, :] = jnp.zeros((n_pad - n, d_in), jnp.float32)

    return pl.pallas_call(
        body,
        out_shape=jax.ShapeDtypeStruct((n_pad, d_in), jnp.float32),
    )(p, dinv, b1, w2)


def _tc_post(p, dinv, b2):
    _, n, d = p.shape

    def body(p_ref, dinv_ref, b_ref, out_ref):
        out_ref[...] = jnp.maximum(
            (p_ref[0] + p_ref[1]) * dinv_ref[...] + b_ref[...], 0.0)

    return pl.pallas_call(
        body,
        out_shape=jax.ShapeDtypeStruct((n, d), jnp.float32),
    )(p, dinv, b2)


def kernel(x, edge_index, W1, b1, W2, b2):
    n, d_in = x.shape
    d_hid = W1.shape[1]
    e = edge_index.shape[1]

    info = plsc.get_sparse_core_info()
    nc, ns = info.num_cores, info.num_subcores
    nw = nc * ns
    per_w = e // nw
    assert per_w * nw == e

    chunk = 128
    per_w_pad = -(-per_w // (4 * chunk)) * (4 * chunk)
    e_pad = per_w_pad * nw
    n_pad = n + 128

    ei = edge_index.astype(jnp.int32)
    pad = n + (jnp.arange(e_pad - e, dtype=jnp.int32) % 128)
    src_r = jnp.concatenate([ei[0], pad]).reshape(nw, per_w_pad // chunk,
                                                  chunk)
    dst_r = jnp.concatenate([ei[1], pad]).reshape(nw, per_w_pad // chunk,
                                                  chunk)
    dst_flat = ei[1].reshape(nw, per_w)

    deg_parts = _make_deg_kernel(n, e, nc, ns)(dst_flat)
    g1, dinv = _tc_pre(deg_parts.T, x, W1, n_pad)

    zeros1 = jnp.zeros((n // ns, d_hid), jnp.float32)
    p1 = _make_agg_kernel(n, n_pad, e_pad, d_hid, nc, ns, chunk, True)(
        g1, src_r, dst_r, zeros1)

    g2 = _tc_mid(p1, dinv, b1.reshape(1, d_hid), W2, n_pad)

    zeros2 = jnp.zeros((n // ns, d_in), jnp.float32)
    p2 = _make_agg_kernel(n, n_pad, e_pad, d_in, nc, ns, chunk, True)(
        g2, src_r, dst_r, zeros2)

    return _tc_post(p2, dinv, b2.reshape(1, d_in))
